# baseline (speedup 1.0000x reference)
import functools

import jax
import jax.numpy as jnp
from jax import lax
from jax.experimental import pallas as pl
from jax.experimental.pallas import tpu as pltpu
from jax.experimental.pallas import tpu_sc as plsc

NC, NS, L = 2, 16, 16
BLK = 32


def _pe_table(size, d):
    pos = jnp.arange(size, dtype=jnp.float32)[:, None]
    div = jnp.power(10000.0, 2.0 * jnp.arange(d, dtype=jnp.float32)[None, :] / float(d))
    pe = pos / div
    pe = pe.at[:, 0::2].set(jnp.sin(pe[:, 0::2]))
    pe = pe.at[:, 1::2].set(jnp.cos(pe[:, 1::2]))
    return pe


def _sc_weights(label, B, T, V, K, nchunks=1, ci=0):
    NW = NC * NS
    Bc = B // nchunks
    rpt = Bc // NW
    nblk = rpt // BLK
    mesh = plsc.VectorSubcoreMesh(core_axis_name="c", subcore_axis_name="s")

    assert K % L == 0 and nblk % 4 == 0

    @functools.partial(
        pl.kernel,
        out_type=jax.ShapeDtypeStruct((Bc, K), jnp.float32),
        mesh=mesh,
        compiler_params=pltpu.CompilerParams(needs_layout_passes=False),
        scratch_types=(
            [pltpu.VMEM((BLK, T), jnp.int32)] * 4
            + [pltpu.VMEM((BLK, K), jnp.float32)] * 2
            + [pltpu.SemaphoreType.DMA] * 6
        ),
    )
    def hist_kernel(label_hbm, out_hbm, lb0, lb1, lb2, lb3, h0, h1,
                    sl0, sl1, sl2, sl3, so0, so1):
        c = lax.axis_index("c")
        s = lax.axis_index("s")
        wid = s * NC + c
        row0 = wid * rpt
        lane = lax.iota(jnp.int32, L)
        lane0 = lane * 0
        zeros16 = jnp.zeros((L,), jnp.float32)
        ones16 = jnp.ones((L,), jnp.float32)
        plus16 = ones16
        minus16 = -ones16
        lbls = [lb0, lb1, lb2, lb3]
        sls = [sl0, sl1, sl2, sl3]
        hs = [h0, h1]
        sos = [so0, so1]

        def lbl_slice(kb):
            return label_hbm.at[pl.ds(ci * Bc + row0 + kb * BLK, BLK), :]

        def out_slice(kb):
            return out_hbm.at[pl.ds(row0 + kb * BLK, BLK), :]

        def ini(j, cc):
            val = jnp.where(j * L + lane < V, zeros16, ones16)
            for i in range(BLK):
                h0[i, pl.ds(j * L, L)] = val
                h1[i, pl.ds(j * L, L)] = val
            return cc

        lax.fori_loop(0, K // L, ini, 0)

        assert (T // 8) * 3 == K // L

        def combined(lbl_v, hist_a, hist_b):
            @plsc.parallel_loop(0, T // 8, 1, unroll=1)
            def _(i):
                for u in range(8):
                    t = i * 8 + u
                    for g in range(BLK // L):
                        row = lane + g * L
                        lv = plsc.load_gather(lbl_v, [row, lane0 + t])
                        m = lv != 0
                        idx = jnp.where(m, lv, V + t)
                        val = jnp.where(m, plus16, minus16)
                        plsc.addupdate_scatter(hist_a, [row, idx], val)
                for q in range(3):
                    jj = i * 3 + q
                    val = jnp.where(jj * L + lane < V, zeros16, ones16)
                    for r in range(BLK):
                        hist_b[r, pl.ds(jj * L, L)] = val

        pltpu.async_copy(lbl_slice(0), lb0, sl0)

        def stage(kb, j):
            lbl_v, sem_l = lbls[j], sls[j]
            hist_a, sem_a = hs[j % 2], sos[j % 2]
            hist_b, sem_b = hs[(j + 1) % 2], sos[(j + 1) % 2]

            pltpu.make_async_copy(lbl_slice(kb), lbl_v, sem_l).wait()

            @pl.when(kb + 1 < nblk)
            def _():
                pltpu.async_copy(lbl_slice(kb + 1), lbls[(j + 1) % 4],
                                 sls[(j + 1) % 4])

            @pl.when(kb >= 1)
            def _():
                pltpu.make_async_copy(hist_b, out_slice(kb - 1), sem_b).wait()

            combined(lbl_v, hist_a, hist_b)
            pltpu.async_copy(hist_a, out_slice(kb), sem_a)

        def quad(p, carry):
            kb = p * 4
            for j in range(4):
                stage(kb + j, j)
            return carry

        lax.fori_loop(0, nblk // 4, quad, 0)
        pltpu.make_async_copy(hs[(nblk - 1) % 2], out_slice(nblk - 1),
                              sos[(nblk - 1) % 2]).wait()

    return hist_kernel(label)


def _tc_matmul(weights, aug, B, K, D):
    BB = 2048

    def mm(wt_ref, tab_ref, out_ref):
        out_ref[...] = jnp.dot(wt_ref[...], tab_ref[...],
                               preferred_element_type=jnp.float32)

    return pl.pallas_call(
        mm,
        grid=(B // BB,),
        in_specs=[
            pl.BlockSpec((BB, K), lambda i: (i, 0)),
            pl.BlockSpec((K, D), lambda i: (0, 0)),
        ],
        out_specs=pl.BlockSpec((BB, D), lambda i: (i, 0)),
        out_shape=jax.ShapeDtypeStruct((B, D), jnp.float32),
    )(weights, aug)


def kernel(label, table):
    B, T = label.shape
    V, D = table.shape
    K = V + T
    NCHUNK = 1
    label = label.astype(jnp.int32)
    aug = jnp.concatenate([table.astype(jnp.float32), _pe_table(T, D)], axis=0)
    outs = []
    for ci in range(NCHUNK):
        w = _sc_weights(label, B, T, V, K, NCHUNK, ci)
        outs.append(_tc_matmul(w, aug, B // NCHUNK, K, D))
    out = jnp.concatenate(outs, axis=0)
    return out[:, None, :]


# speedup vs baseline: 1.6294x; 1.6294x over previous
import functools

import jax
import jax.numpy as jnp
from jax import lax
from jax.experimental import pallas as pl
from jax.experimental.pallas import tpu as pltpu
from jax.experimental.pallas import tpu_sc as plsc

NC, NS, L = 2, 16, 16
BLK = 32


def _pe_table(size, d):
    pos = jnp.arange(size, dtype=jnp.float32)[:, None]
    div = jnp.power(10000.0, 2.0 * jnp.arange(d, dtype=jnp.float32)[None, :] / float(d))
    pe = pos / div
    pe = pe.at[:, 0::2].set(jnp.sin(pe[:, 0::2]))
    pe = pe.at[:, 1::2].set(jnp.cos(pe[:, 1::2]))
    return pe


def _sc_weights(label, B, T, V, K, nchunks=1, ci=0):
    NW = NC * NS
    Bc = B // nchunks
    rpt = Bc // NW
    nblk = rpt // BLK
    mesh = plsc.VectorSubcoreMesh(core_axis_name="c", subcore_axis_name="s")

    assert K % L == 0 and nblk % 4 == 0

    @functools.partial(
        pl.kernel,
        out_type=jax.ShapeDtypeStruct((Bc, K), jnp.float32),
        mesh=mesh,
        compiler_params=pltpu.CompilerParams(needs_layout_passes=False),
        scratch_types=(
            [pltpu.VMEM((BLK, T), jnp.int32)] * 4
            + [pltpu.VMEM((BLK, K), jnp.float32)] * 2
            + [pltpu.SemaphoreType.DMA] * 6
        ),
    )
    def hist_kernel(label_hbm, out_hbm, lb0, lb1, lb2, lb3, h0, h1,
                    sl0, sl1, sl2, sl3, so0, so1):
        c = lax.axis_index("c")
        s = lax.axis_index("s")
        wid = s * NC + c
        row0 = wid * rpt
        lane = lax.iota(jnp.int32, L)
        lane0 = lane * 0
        zeros16 = jnp.zeros((L,), jnp.float32)
        ones16 = jnp.ones((L,), jnp.float32)
        plus16 = ones16
        minus16 = -ones16
        lbls = [lb0, lb1, lb2, lb3]
        sls = [sl0, sl1, sl2, sl3]
        hs = [h0, h1]
        sos = [so0, so1]

        def lbl_slice(kb):
            return label_hbm.at[pl.ds(ci * Bc + row0 + kb * BLK, BLK), :]

        def out_slice(kb):
            return out_hbm.at[pl.ds(row0 + kb * BLK, BLK), :]

        def ini(j, cc):
            val = jnp.where(j * L + lane < V, zeros16, ones16)
            for i in range(BLK):
                h0[i, pl.ds(j * L, L)] = val
                h1[i, pl.ds(j * L, L)] = val
            return cc

        lax.fori_loop(0, K // L, ini, 0)

        def scan(lbl_v, hist_v):
            @plsc.parallel_loop(0, T, 1, unroll=8)
            def _(t):
                for g in range(BLK // L):
                    row = lane + g * L
                    lv = plsc.load_gather(lbl_v, [row, lane0 + t])
                    m = lv != 0
                    idx = jnp.where(m, lv, V + t)
                    val = jnp.where(m, plus16, minus16)
                    plsc.addupdate_scatter(hist_v, [row, idx], val)

        pltpu.async_copy(lbl_slice(0), lb0, sl0)

        def stage(kb, j):
            lbl_v, sem_l = lbls[j], sls[j]
            hist_a, sem_a = hs[j % 2], sos[j % 2]

            pltpu.make_async_copy(lbl_slice(kb), lbl_v, sem_l).wait()

            @pl.when(kb + 1 < nblk)
            def _():
                pltpu.async_copy(lbl_slice(kb + 1), lbls[(j + 1) % 4],
                                 sls[(j + 1) % 4])

            @pl.when(kb >= 2)
            def _():
                pltpu.make_async_copy(hist_a, out_slice(kb - 2), sem_a).wait()

            @plsc.parallel_loop(0, K // L, 1, unroll=2)
            def _(jj):
                val = jnp.where(jj * L + lane < V, zeros16, ones16)
                for i in range(BLK):
                    hist_a[i, pl.ds(jj * L, L)] = val

            scan(lbl_v, hist_a)
            pltpu.async_copy(hist_a, out_slice(kb), sem_a)

        def quad(p, carry):
            kb = p * 4
            for j in range(4):
                stage(kb + j, j)
            return carry

        lax.fori_loop(0, nblk // 4, quad, 0)
        pltpu.make_async_copy(h0, out_slice(nblk - 2), so0).wait()
        pltpu.make_async_copy(h1, out_slice(nblk - 1), so1).wait()

    return hist_kernel(label)


def _tc_matmul(weights, aug, B, K, D):
    BB = 2048

    def mm(wt_ref, tab_ref, out_ref):
        out_ref[...] = jnp.dot(wt_ref[...], tab_ref[...],
                               preferred_element_type=jnp.float32)

    return pl.pallas_call(
        mm,
        grid=(B // BB,),
        in_specs=[
            pl.BlockSpec((BB, K), lambda i: (i, 0)),
            pl.BlockSpec((K, D), lambda i: (0, 0)),
        ],
        out_specs=pl.BlockSpec((BB, D), lambda i: (i, 0)),
        out_shape=jax.ShapeDtypeStruct((B, D), jnp.float32),
    )(weights, aug)


def kernel(label, table):
    B, T = label.shape
    V, D = table.shape
    K = V + T
    NCHUNK = 1
    label = label.astype(jnp.int32)
    aug = jnp.concatenate([table.astype(jnp.float32), _pe_table(T, D)], axis=0)
    outs = []
    for ci in range(NCHUNK):
        w = _sc_weights(label, B, T, V, K, NCHUNK, ci)
        outs.append(_tc_matmul(w, aug, B // NCHUNK, K, D))
    out = jnp.concatenate(outs, axis=0)
    return out[:, None, :]
